# baseline (device time: 33825 ns/iter reference)
import jax
import jax.numpy as jnp
from jax import lax
from jax.experimental import pallas as pl
from jax.experimental.pallas import tpu as pltpu

N_DEV = 8
B = 2
SQ = 128
D_MODEL = 512
HQ_LOCAL = 4
DH = 64
HD = HQ_LOCAL * DH
SKV_LOC = 128
SKV = SKV_LOC * N_DEV
BLK = 64

QBLOCKS = {0: (0, 3, 6, 9, 12, 15), 1: (0, 1, 2, 5, 8, 11, 14)}
BFLY = (1, 3, 4)


def kernel(x, Wq, K_ext, V_ext, Wo):
    k2 = K_ext.reshape(B, SKV_LOC, N_DEV * HD).astype(jnp.bfloat16)
    v2 = V_ext.reshape(B, SKV_LOC, N_DEV * HD).astype(jnp.bfloat16)
    x16 = x.astype(jnp.bfloat16)
    wq16 = Wq.astype(jnp.bfloat16)
    wo16 = Wo.astype(jnp.bfloat16)

    def body(
        x_ref, wq_ref, k_ref, v_ref, wo_ref, out_ref,
        kgath, vgath, pacc, pstage, pin,
        ksend_sems, krecv_sems, vsend_sems, vrecv_sems,
        psend_sems, precv_sems,
    ):
        my = lax.axis_index("i")

        barrier_sem = pltpu.get_barrier_semaphore()
        for d in range(1, N_DEV):
            peer = (my + d) % N_DEV
            pl.semaphore_signal(
                barrier_sem, inc=1, device_id=(peer,),
                device_id_type=pl.DeviceIdType.MESH,
            )
        pl.semaphore_wait(barrier_sem, N_DEV - 1)

        my_off = my * SKV_LOC
        k_rdmas = []
        v_rdmas = []
        for d in range(1, N_DEV):
            tgt = (my + d) % N_DEV
            lstart = tgt * HD
            kr = pltpu.make_async_remote_copy(
                src_ref=k_ref.at[:, :, pl.ds(lstart, HD)],
                dst_ref=kgath.at[:, pl.ds(my_off, SKV_LOC), :],
                send_sem=ksend_sems.at[d - 1],
                recv_sem=krecv_sems.at[d - 1],
                device_id=(tgt,),
                device_id_type=pl.DeviceIdType.MESH,
            )
            kr.start()
            k_rdmas.append(kr)
        for d in range(1, N_DEV):
            tgt = (my + d) % N_DEV
            lstart = tgt * HD
            vr = pltpu.make_async_remote_copy(
                src_ref=v_ref.at[:, :, pl.ds(lstart, HD)],
                dst_ref=vgath.at[:, pl.ds(my_off, SKV_LOC), :],
                send_sem=vsend_sems.at[d - 1],
                recv_sem=vrecv_sems.at[d - 1],
                device_id=(tgt,),
                device_id_type=pl.DeviceIdType.MESH,
            )
            vr.start()
            v_rdmas.append(vr)

        wq = wq_ref[:, :]
        qs = [
            jax.lax.dot(
                x_ref[b], wq, preferred_element_type=jnp.float32
            ).astype(jnp.bfloat16)
            for b in range(B)
        ]

        my_l = my * HD
        kgath[:, pl.ds(my_off, SKV_LOC), :] = k_ref[:, :, pl.ds(my_l, HD)]
        vgath[:, pl.ds(my_off, SKV_LOC), :] = v_ref[:, :, pl.ds(my_l, HD)]

        for d in range(1, N_DEV):
            k_rdmas[d - 1].wait_recv()

        weights = {}
        for b in range(B):
            for qb in range(2):
                blocks = QBLOCKS[qb]
                k_rows = jnp.concatenate(
                    [kgath[b, kb * BLK:(kb + 1) * BLK, :] for kb in blocks],
                    axis=0,
                )
                ws = []
                for h in range(HQ_LOCAL):
                    q_bh = qs[b][qb * BLK:(qb + 1) * BLK, h * DH:(h + 1) * DH]
                    k_sub = k_rows[:, h * DH:(h + 1) * DH]
                    s = lax.dot_general(
                        q_bh, k_sub, (((1,), (1,)), ((), ())),
                        preferred_element_type=jnp.float32,
                    ) * 0.125
                    m = jnp.max(s, axis=1, keepdims=True)
                    w = jnp.exp(s - m)
                    w = w / jnp.sum(w, axis=1, keepdims=True)
                    ws.append(w.astype(jnp.bfloat16))
                weights[(b, qb)] = ws

        for d in range(1, N_DEV):
            v_rdmas[d - 1].wait_recv()

        wo = wo_ref[:, :]

        def attend_quarter(b, qb):
            blocks = QBLOCKS[qb]
            v_rows = jnp.concatenate(
                [vgath[b, kb * BLK:(kb + 1) * BLK, :] for kb in blocks],
                axis=0,
            )
            ctxs = [
                jax.lax.dot(
                    weights[(b, qb)][h], v_rows[:, h * DH:(h + 1) * DH],
                    preferred_element_type=jnp.float32,
                )
                for h in range(HQ_LOCAL)
            ]
            ctx_row = jnp.concatenate(ctxs, axis=1).astype(
                jnp.bfloat16
            )
            pacc[pl.ds(b * SQ + qb * BLK, BLK), :] = jax.lax.dot(
                ctx_row, wo, preferred_element_type=jnp.float32
            )

        NQ = 4

        def bfly_start(step, q):
            partner = lax.bitwise_xor(my, BFLY[step])
            pstage[step, pl.ds(q * BLK, BLK), :] = pacc[
                pl.ds(q * BLK, BLK), :
            ].astype(jnp.bfloat16)
            pr = pltpu.make_async_remote_copy(
                src_ref=pstage.at[step, pl.ds(q * BLK, BLK), :],
                dst_ref=pin.at[step, pl.ds(q * BLK, BLK), :],
                send_sem=psend_sems.at[step, q],
                recv_sem=precv_sems.at[step, q],
                device_id=(partner,),
                device_id_type=pl.DeviceIdType.MESH,
            )
            pr.start()
            return pr

        def bfly_finish(pr, step, q):
            pr.wait_send()
            pr.wait_recv()
            pacc[pl.ds(q * BLK, BLK), :] = (
                pacc[pl.ds(q * BLK, BLK), :] + pin[step, pl.ds(q * BLK, BLK), :]
            )

        prs = [[None] * NQ for _ in range(len(BFLY))]
        for q, (b, qb) in enumerate([(0, 0), (0, 1), (1, 0), (1, 1)]):
            attend_quarter(b, qb)
            prs[0][q] = bfly_start(0, q)
        for step in range(len(BFLY)):
            for q in range(NQ):
                bfly_finish(prs[step][q], step, q)
                if step + 1 < len(BFLY):
                    prs[step + 1][q] = bfly_start(step + 1, q)
        out_ref[0, :, :] = pacc[pl.ds(0, SQ), :]
        out_ref[1, :, :] = pacc[pl.ds(SQ, SQ), :]

        for d in range(1, N_DEV):
            k_rdmas[d - 1].wait_send()
            v_rdmas[d - 1].wait_send()

    return pl.pallas_call(
        body,
        out_shape=jax.ShapeDtypeStruct((B, SQ, D_MODEL), jnp.float32),
        in_specs=[pl.BlockSpec(memory_space=pltpu.VMEM)] * 5,
        out_specs=pl.BlockSpec(memory_space=pltpu.VMEM),
        scratch_shapes=[
            pltpu.VMEM((B, SKV, HD), jnp.bfloat16),
            pltpu.VMEM((B, SKV, HD), jnp.bfloat16),
            pltpu.VMEM((B * SQ, D_MODEL), jnp.float32),
            pltpu.VMEM((len(BFLY), B * SQ, D_MODEL), jnp.bfloat16),
            pltpu.VMEM((len(BFLY), B * SQ, D_MODEL), jnp.bfloat16),
            pltpu.SemaphoreType.DMA((N_DEV - 1,)),
            pltpu.SemaphoreType.DMA((N_DEV - 1,)),
            pltpu.SemaphoreType.DMA((N_DEV - 1,)),
            pltpu.SemaphoreType.DMA((N_DEV - 1,)),
            pltpu.SemaphoreType.DMA((len(BFLY), 4)),
            pltpu.SemaphoreType.DMA((len(BFLY), 4)),
        ],
        compiler_params=pltpu.CompilerParams(collective_id=0),
    )(x16, wq16, k2, v2, wo16)
